# Initial kernel scaffold; baseline (speedup 1.0000x reference)
import functools

import jax
import jax.numpy as jnp
from jax import lax
from jax.experimental import pallas as pl
from jax.experimental.pallas import tpu as pltpu
from jax.experimental.pallas import tpu_sc as plsc

NC = 2
NS = 16
NW = NC * NS
LANES = 128


def _sc_mesh():
    return plsc.VectorSubcoreMesh(core_axis_name="c", subcore_axis_name="s")


def _make_degree_kernel(NP, H, S, rps):
    @functools.partial(
        pl.kernel,
        out_type=jax.ShapeDtypeStruct((NC, NP, H), jnp.float32),
        mesh=_sc_mesh(),
        scratch_types=[
            pltpu.VMEM((S, LANES), jnp.int32),
            pltpu.VMEM((LANES, H), jnp.float32),
            pltpu.VMEM_SHARED((NP, H), jnp.float32),
            pltpu.SemaphoreType.DMA,
        ],
    )
    def deg_kernel(dst_hbm, ones_hbm, zeros_hbm, out_hbm, didx, onesv, acc, sem):
        c = lax.axis_index("c")
        s = lax.axis_index("s")
        w = c * NS + s
        pltpu.sync_copy(dst_hbm.at[w], didx)
        pltpu.sync_copy(ones_hbm, onesv)
        pltpu.sync_copy(
            zeros_hbm.at[pl.ds(s * rps, rps)], acc.at[pl.ds(s * rps, rps)]
        )
        plsc.subcore_barrier()

        @pl.loop(0, S)
        def _(j):
            pltpu.sync_copy(onesv, acc.at[didx.at[j]], add=True)

        plsc.subcore_barrier()
        pltpu.sync_copy(
            acc.at[pl.ds(s * rps, rps)], out_hbm.at[c, pl.ds(s * rps, rps)]
        )

    return deg_kernel


def _make_agg_kernel(NP, H, S, rps):
    @functools.partial(
        pl.kernel,
        out_type=jax.ShapeDtypeStruct((NC, NP, H), jnp.float32),
        mesh=_sc_mesh(),
        scratch_types=[
            pltpu.VMEM((S, LANES), jnp.int32),
            pltpu.VMEM((S, LANES), jnp.int32),
            pltpu.VMEM((LANES, H), jnp.float32),
            pltpu.VMEM_SHARED((NP, H), jnp.float32),
            pltpu.SemaphoreType.DMA,
        ],
    )
    def agg_kernel(g_hbm, src_hbm, dst_hbm, zeros_hbm, out_hbm,
                   sidx, didx, rows, acc, sem):
        c = lax.axis_index("c")
        s = lax.axis_index("s")
        w = c * NS + s
        pltpu.sync_copy(src_hbm.at[w], sidx)
        pltpu.sync_copy(dst_hbm.at[w], didx)
        pltpu.sync_copy(
            zeros_hbm.at[pl.ds(s * rps, rps)], acc.at[pl.ds(s * rps, rps)]
        )
        plsc.subcore_barrier()

        @pl.loop(0, S)
        def _(j):
            pltpu.async_copy(g_hbm.at[sidx.at[j]], rows, sem).wait()
            pltpu.sync_copy(rows, acc.at[didx.at[j]], add=True)

        plsc.subcore_barrier()
        pltpu.sync_copy(
            acc.at[pl.ds(s * rps, rps)], out_hbm.at[c, pl.ds(s * rps, rps)]
        )

    return agg_kernel


def _tc_matmul1(x, W1):
    def body(x_ref, w_ref, h_ref):
        h_ref[...] = jnp.dot(
            x_ref[...], w_ref[...], preferred_element_type=jnp.float32
        )

    return pl.pallas_call(
        body,
        out_shape=jax.ShapeDtypeStruct((x.shape[0], W1.shape[1]), jnp.float32),
    )(x, W1)


def _tc_scale1(h, degp, NP):
    N, H = h.shape

    def body(h_ref, degp_ref, g_ref):
        deg = degp_ref[0][:, 0:1] + degp_ref[1][:, 0:1] + 1.0
        dinv = lax.rsqrt(deg)
        g_ref[...] = jnp.zeros((NP, H), jnp.float32)
        g_ref[0:N, :] = h_ref[...] * dinv[0:N]

    return pl.pallas_call(
        body,
        out_shape=jax.ShapeDtypeStruct((NP, H), jnp.float32),
    )(h, degp)


def _tc_mid(degp, agg1, g1, b1, N):
    NP, H = g1.shape

    def body(degp_ref, agg_ref, g1_ref, b1_ref, g2_ref):
        deg = degp_ref[0][:, 0:1] + degp_ref[1][:, 0:1] + 1.0
        dinv = lax.rsqrt(deg)
        agg = agg_ref[0] + agg_ref[1]
        z = jnp.maximum(dinv * (agg + g1_ref[...]) + b1_ref[...], 0.0)
        rows = lax.broadcasted_iota(jnp.int32, (NP, 1), 0)
        g2_ref[...] = jnp.where(rows < N, dinv * z, 0.0)

    return pl.pallas_call(
        body,
        out_shape=jax.ShapeDtypeStruct((NP, H), jnp.float32),
    )(degp, agg1, g1, b1)


def _tc_final(degp, agg2, g2, W2, b2, N):
    NP, H = g2.shape
    C = W2.shape[1]

    def body(degp_ref, agg_ref, g2_ref, w2_ref, b2_ref, out_ref):
        deg = degp_ref[0][:, 0:1] + degp_ref[1][:, 0:1] + 1.0
        dinv = lax.rsqrt(deg)
        y = dinv * (agg_ref[0] + agg_ref[1] + g2_ref[...])
        out_ref[...] = (
            jnp.dot(y[0:N, :], w2_ref[...], preferred_element_type=jnp.float32)
            + b2_ref[...]
        )

    return pl.pallas_call(
        body,
        out_shape=jax.ShapeDtypeStruct((N, C), jnp.float32),
    )(degp, agg2, g2, W2, b2)


def kernel(x, edge_index, W1, b1, W2, b2):
    N, D = x.shape
    H = W1.shape[1]
    C = W2.shape[1]
    E = edge_index.shape[1]

    NP = ((N + 1 + NS - 1) // NS) * NS
    rps = NP // NS

    S = (E + NW * LANES - 1) // (NW * LANES)
    EP = NW * LANES * S

    src = edge_index[0].astype(jnp.int32)
    dst = edge_index[1].astype(jnp.int32)
    pad = jnp.full((EP - E,), N, dtype=jnp.int32)
    src3 = jnp.concatenate([src, pad]).reshape(NW, S, LANES)
    dst3 = jnp.concatenate([dst, pad]).reshape(NW, S, LANES)

    zeros = jnp.zeros((NP, H), jnp.float32)
    ones = jnp.ones((LANES, H), jnp.float32)

    deg_kernel = _make_degree_kernel(NP, H, S, rps)
    agg_kernel = _make_agg_kernel(NP, H, S, rps)

    degp = deg_kernel(dst3, ones, zeros)
    h1 = _tc_matmul1(x, W1)
    g1 = _tc_scale1(h1, degp, NP)
    agg1 = agg_kernel(g1, src3, dst3, zeros)
    g2 = _tc_mid(degp, agg1, g1, b1.reshape(1, H), N)
    agg2 = agg_kernel(g2, src3, dst3, zeros)
    out = _tc_final(degp, agg2, g2, W2, b2.reshape(1, C), N)
    return out


# speedup vs baseline: 105.8149x; 105.8149x over previous
import functools

import jax
import jax.numpy as jnp
from jax import lax
from jax.experimental import pallas as pl
from jax.experimental.pallas import tpu as pltpu
from jax.experimental.pallas import tpu_sc as plsc

NC = 2
NS = 16
NW = NC * NS
LANES = 128


def _sc_mesh():
    return plsc.VectorSubcoreMesh(core_axis_name="c", subcore_axis_name="s")


_SC_PARAMS = pltpu.CompilerParams(use_tc_tiling_on_sc=False)


def _make_degree_kernel(NP, H, S, rps):
    @functools.partial(
        pl.kernel,
        out_type=jax.ShapeDtypeStruct((NC, NP, H), jnp.float32),
        mesh=_sc_mesh(),
        compiler_params=_SC_PARAMS,
        scratch_types=[
            pltpu.VMEM((S, LANES), jnp.int32),
            pltpu.VMEM((LANES, H), jnp.float32),
            pltpu.VMEM_SHARED((NP, H), jnp.float32),
            [pltpu.SemaphoreType.DMA] * NBUF,
        ],
    )
    def deg_kernel(dst_hbm, ones_hbm, zeros_hbm, out_hbm, didx, onesv, acc, sem):
        c = lax.axis_index("c")
        s = lax.axis_index("s")
        w = c * NS + s
        pltpu.sync_copy(dst_hbm.at[w], didx)
        pltpu.sync_copy(ones_hbm, onesv)
        pltpu.sync_copy(
            zeros_hbm.at[pl.ds(s * rps, rps)], acc.at[pl.ds(s * rps, rps)]
        )
        plsc.subcore_barrier()

        for b in range(NBUF):
            pltpu.async_copy(onesv, acc.at[didx.at[b]], sem[b], add=True)

        @pl.loop(0, S, step=NBUF)
        def _(j):
            for b in range(NBUF):
                pltpu.make_async_copy(onesv, acc.at[didx.at[j + b]], sem[b]).wait()

                @pl.when(j + b + NBUF < S)
                def _():
                    pltpu.async_copy(
                        onesv, acc.at[didx.at[j + b + NBUF]], sem[b], add=True
                    )

        plsc.subcore_barrier()
        pltpu.sync_copy(
            acc.at[pl.ds(s * rps, rps)], out_hbm.at[c, pl.ds(s * rps, rps)]
        )

    return deg_kernel


NBUF = 4
CH = 16


def _make_agg_kernel(NP, H, S, rps):
    assert S % CH == 0
    nch = S // CH

    @functools.partial(
        pl.kernel,
        out_type=jax.ShapeDtypeStruct((NC, NP, H), jnp.float32),
        mesh=_sc_mesh(),
        compiler_params=_SC_PARAMS,
        scratch_types=[
            pltpu.VMEM((S // CH, CH * LANES), jnp.int32),
            pltpu.VMEM((S // CH, CH * LANES), jnp.int32),
            pltpu.VMEM((2, CH * LANES, H), jnp.float32),
            pltpu.VMEM_SHARED((NP, H), jnp.float32),
            pltpu.VMEM_SHARED((NP, H), jnp.float32),
            [pltpu.SemaphoreType.DMA] * 2,
            [pltpu.SemaphoreType.DMA] * 2,
        ],
    )
    def agg_kernel(g_hbm, src_hbm, dst_hbm, zeros_hbm, out_hbm,
                   sidx, didx, rows, gsh, acc, gsem, ssem):
        c = lax.axis_index("c")
        s = lax.axis_index("s")
        w = c * NS + s
        sl = pl.ds(s * rps, rps)
        pltpu.sync_copy(src_hbm.at[w], sidx)
        pltpu.sync_copy(dst_hbm.at[w], didx)
        pltpu.sync_copy(g_hbm.at[sl], gsh.at[sl])
        pltpu.sync_copy(zeros_hbm.at[sl], acc.at[sl])
        plsc.subcore_barrier()

        def start_gather(k):
            pltpu.async_copy(gsh.at[sidx.at[k]], rows.at[k % 2], gsem[k % 2])

        def wait_gather(k):
            pltpu.make_async_copy(gsh.at[sidx.at[k]], rows.at[k % 2],
                                  gsem[k % 2]).wait()

        def start_scatter(k):
            pltpu.async_copy(rows.at[k % 2], acc.at[didx.at[k]],
                             ssem[k % 2], add=True)

        def wait_scatter(k):
            pltpu.make_async_copy(rows.at[k % 2], acc.at[didx.at[k]],
                                  ssem[k % 2]).wait()

        start_gather(0)
        if nch > 1:
            start_gather(1)
        for k in range(nch):
            wait_gather(k)
            start_scatter(k)
            if k >= 1 and k + 1 < nch:
                wait_scatter(k - 1)
                start_gather(k + 1)
        if nch > 1:
            wait_scatter(nch - 2)
        wait_scatter(nch - 1)

        plsc.subcore_barrier()
        pltpu.sync_copy(acc.at[sl], out_hbm.at[c, sl])

    return agg_kernel


def _tc_prep(edge_index, N, NP, EP):
    E = edge_index.shape[1]

    def body(e_ref, s_ref, d_ref):
        pad = N + lax.rem(
            lax.broadcasted_iota(jnp.int32, (EP - E,), 0), NP - N
        )
        s_ref[0:E] = e_ref[0, :]
        s_ref[E:EP] = pad
        d_ref[0:E] = e_ref[1, :]
        d_ref[E:EP] = pad

    return pl.pallas_call(
        body,
        out_shape=[jax.ShapeDtypeStruct((EP,), jnp.int32)] * 2,
    )(edge_index)


def _tc_matmul1(x, W1, NP):
    N = x.shape[0]
    H = W1.shape[1]

    def body(x_ref, w_ref, h_ref):
        h_ref[N:NP, :] = jnp.zeros((NP - N, H), jnp.float32)
        h_ref[0:N, :] = jnp.dot(
            x_ref[...], w_ref[...], preferred_element_type=jnp.float32
        )

    return pl.pallas_call(
        body,
        out_shape=jax.ShapeDtypeStruct((NP, H), jnp.float32),
    )(x, W1)


def _tc_gscale(degp128, h128):
    R = h128.shape[0]

    def body(degp_ref, h_ref, g_ref):
        deg = degp_ref[0:R] + degp_ref[R:2 * R] + 1.0
        g_ref[...] = lax.rsqrt(deg) * h_ref[...]

    return pl.pallas_call(
        body,
        out_shape=jax.ShapeDtypeStruct((R, 128), jnp.float32),
    )(degp128, h128)


def _tc_mid(degp128, agg128, g1_128, b1row, NP, H):
    R = NP * H // 128

    def body(degp_ref, agg_ref, g1_ref, b1_ref, g2_ref):
        deg = degp_ref[0:R] + degp_ref[R:2 * R] + 1.0
        dinv = lax.rsqrt(deg)
        agg = agg_ref[0:R] + agg_ref[R:2 * R] + g1_ref[...]
        z = jnp.maximum(dinv * agg + b1_ref[...], 0.0)
        g2_ref[...] = dinv * z

    return pl.pallas_call(
        body,
        out_shape=jax.ShapeDtypeStruct((R, 128), jnp.float32),
    )(degp128, agg128, g1_128, b1row)


def _tc_final(degp128, agg2_128, g2_128, W2big, b2row, NP, H):
    R = NP * H // 128

    def body(degp_ref, agg_ref, g2_ref, w2b_ref, b2_ref, out_ref):
        deg = degp_ref[0:R] + degp_ref[R:2 * R] + 1.0
        dinv = lax.rsqrt(deg)
        y = dinv * (agg_ref[0:R] + agg_ref[R:2 * R] + g2_ref[...])
        out_ref[...] = (
            jnp.dot(y, w2b_ref[...], preferred_element_type=jnp.float32)
            + b2_ref[...]
        )

    return pl.pallas_call(
        body,
        out_shape=jax.ShapeDtypeStruct((R, W2big.shape[1]), jnp.float32),
    )(degp128, agg2_128, g2_128, W2big, b2row)


def kernel(x, edge_index, W1, b1, W2, b2):
    N, D = x.shape
    H = W1.shape[1]
    C = W2.shape[1]
    E = edge_index.shape[1]

    NP = ((N + 1 + NS * 8 - 1) // (NS * 8)) * (NS * 8)
    rps = NP // NS

    S = (E + NW * LANES - 1) // (NW * LANES)
    S = ((S + CH - 1) // CH) * CH
    EP = NW * LANES * S

    srcp, dstp = _tc_prep(edge_index.astype(jnp.int32), N, NP, EP)
    dst3 = dstp.reshape(NW, S, LANES)
    src3c = srcp.reshape(NW, S // CH, CH * LANES)
    dst3c = dstp.reshape(NW, S // CH, CH * LANES)

    zeros = jnp.zeros((NP, H), jnp.float32)
    ones = jnp.ones((LANES, H), jnp.float32)
    rep = 128 // H
    R = NP * H // 128
    b1row = jnp.tile(b1.astype(jnp.float32), rep).reshape(1, 128)
    W2big = jnp.kron(jnp.eye(rep, dtype=jnp.float32), W2)
    b2row = jnp.tile(b2.astype(jnp.float32), rep).reshape(1, rep * C)

    deg_kernel = _make_degree_kernel(NP, H, S, rps)
    agg_kernel = _make_agg_kernel(NP, H, S, rps)

    h = _tc_matmul1(x, W1, NP)
    h128 = h.reshape(R, 128)
    degp = deg_kernel(dst3, ones, zeros)
    degp128 = degp.reshape(2 * R, 128)
    g1f = _tc_gscale(degp128, h128)
    agg1 = agg_kernel(g1f.reshape(NP, H), src3c, dst3c, zeros)
    g2_128 = _tc_mid(degp128, agg1.reshape(2 * R, 128), g1f, b1row, NP, H)
    agg2 = agg_kernel(g2_128.reshape(NP, H), src3c, dst3c, zeros)
    out128 = _tc_final(degp128, agg2.reshape(2 * R, 128), g2_128,
                       W2big, b2row, NP, H)
    return out128.reshape(NP, C)[0:N]
